# baseline (speedup 1.0000x reference)
import functools

import jax
import jax.numpy as jnp
from jax import lax
from jax.experimental import pallas as pl
from jax.experimental.pallas import tpu as pltpu
from jax.experimental.pallas import tpu_sc as plsc

EMB = 128
INT = 64
M = 160000
K = 320000

BM = 2000
BK = 4000


def _silu(v):
    return v * jax.nn.sigmoid(v)


def _dotb(a, b):
    return jnp.dot(a.astype(jnp.bfloat16), b.astype(jnp.bfloat16),
                   preferred_element_type=jnp.float32)


def _stage1_body(x_ref, rbf_ref, wji_ref, bji_ref, wkj_ref, bkj_ref,
                 wrbf_ref, wdown_ref, xji_out, t_out):
    xb = x_ref[...]
    x_ji = _silu(_dotb(xb, wji_ref[...]) + bji_ref[...])
    x_kj = _silu(_dotb(xb, wkj_ref[...]) + bkj_ref[...])
    rbf_e = jnp.dot(rbf_ref[...], wrbf_ref[...],
                    preferred_element_type=jnp.float32)
    x_kj = x_kj * rbf_e
    t = _silu(_dotb(x_kj, wdown_ref[...]))
    xji_out[...] = x_ji
    t_out[...] = t


def _stage1(x, rbf, w_ji, b_ji, w_kj, b_kj, w_rbf, w_down):
    grid = (M // BM,)
    blk = lambda d: pl.BlockSpec((BM, d), lambda i: (i, 0))
    full = lambda a: pl.BlockSpec(a.shape, lambda i: (0,) * a.ndim)
    return pl.pallas_call(
        _stage1_body,
        grid=grid,
        in_specs=[blk(EMB), blk(rbf.shape[1]), full(w_ji), full(b_ji),
                  full(w_kj), full(b_kj), full(w_rbf), full(w_down)],
        out_specs=[blk(EMB), blk(INT)],
        out_shape=[jax.ShapeDtypeStruct((M, EMB), jnp.float32),
                   jax.ShapeDtypeStruct((M, INT), jnp.float32)],
    )(x, rbf, w_ji, b_ji, w_kj, b_kj, w_rbf, w_down)


def _stage2_body(sbf_ref, wsbf_ref, out_ref):
    out_ref[...] = jnp.dot(sbf_ref[...], wsbf_ref[...],
                           preferred_element_type=jnp.float32)


def _stage2(sbf, w_sbf):
    grid = (K // BK,)
    return pl.pallas_call(
        _stage2_body,
        grid=grid,
        in_specs=[pl.BlockSpec((BK, sbf.shape[1]), lambda i: (i, 0)),
                  pl.BlockSpec(w_sbf.shape, lambda i: (0, 0))],
        out_specs=pl.BlockSpec((BK, INT), lambda i: (i, 0)),
        out_shape=jax.ShapeDtypeStruct((K, INT), jnp.float32),
    )(sbf, w_sbf)


def _stage3_body(x_ref, xji_ref, seg_ref, wup_ref,
                 w1_ref, b1_ref, w2_ref, b2_ref, wf_ref, bf_ref,
                 wa01_ref, ba01_ref, wa02_ref, ba02_ref,
                 wa11_ref, ba11_ref, wa12_ref, ba12_ref, out_ref):
    dot = _dotb
    u = _silu(dot(seg_ref[...], wup_ref[...]))
    x2 = xji_ref[...] + u
    h = _silu(dot(x2, w1_ref[...]) + b1_ref[...])
    h = _silu(dot(h, w2_ref[...]) + b2_ref[...])
    x2 = x2 + h
    x2 = _silu(dot(x2, wf_ref[...]) + bf_ref[...])
    out = x_ref[...] + x2
    h = _silu(dot(out, wa01_ref[...]) + ba01_ref[...])
    h = _silu(dot(h, wa02_ref[...]) + ba02_ref[...])
    out = out + h
    h = _silu(dot(out, wa11_ref[...]) + ba11_ref[...])
    h = _silu(dot(h, wa12_ref[...]) + ba12_ref[...])
    out_ref[...] = out + h


def _stage3(x, x_ji, seg, p):
    grid = (M // BM,)
    blk = lambda d: pl.BlockSpec((BM, d), lambda i: (i, 0))
    full = lambda a: pl.BlockSpec(a.shape, lambda i: (0,) * a.ndim)
    b = lambda name: p[name].reshape(1, EMB)
    args = (x, x_ji, seg, p['W_up'],
            p['W_bs0_1'], b('b_bs0_1'), p['W_bs0_2'], b('b_bs0_2'),
            p['W_fbs'], b('b_fbs'),
            p['W_as0_1'], b('b_as0_1'), p['W_as0_2'], b('b_as0_2'),
            p['W_as1_1'], b('b_as1_1'), p['W_as1_2'], b('b_as1_2'))
    return pl.pallas_call(
        _stage3_body,
        grid=grid,
        in_specs=[blk(EMB), blk(EMB), blk(INT)] + [full(a) for a in args[3:]],
        out_specs=blk(EMB),
        out_shape=jax.ShapeDtypeStruct((M, EMB), jnp.float32),
    )(*args)


NC = 2
NS = 16
L = 16
HALF = M // NC
CAP = 20000
NP = HALF // CAP
KS = K // NS
BA = 2000
NBLK = KS // BA
NFILT = BA // L
G = 128
GV = G // L
MUR = 4
ZSTRIPE = CAP // NS
DUMP = CAP


def _sc_body(a0, a1, t, sbf_e, zc, seg,
             a0b0, a1b0, a0b1, a1b1, dst_st, src_st, ang_st,
             dst_ix0, src_ix0, ang_ix0, dst_ix1, src_ix1, ang_ix1,
             tbuf0, sbuf0, tbuf1, sbuf1, table,
             isem0, isem1, gsem0, gsem1):
    c = lax.axis_index("c")
    s = lax.axis_index("s")
    kbase = s * KS

    def fire_idx(b, a0blk, a1blk, isem):
        kb = kbase + b * BA
        pltpu.async_copy(a0.at[pl.ds(kb, BA)], a0blk, isem)
        pltpu.async_copy(a1.at[pl.ds(kb, BA)], a1blk, isem)

    def wait_idx(a0blk, a1blk, isem):
        pltpu.make_async_copy(a0.at[pl.ds(0, BA)], a0blk, isem).wait()
        pltpu.make_async_copy(a1.at[pl.ds(0, BA)], a1blk, isem).wait()

    def fire_grp(j, dix, six, aix, tb, sb, gsem):
        for v in range(GV):
            sl = pl.ds(j * G + v * L, L)
            dix[0, pl.ds(v * L, L)] = dst_st[sl]
            six[0, pl.ds(v * L, L)] = src_st[sl]
            aix[0, pl.ds(v * L, L)] = ang_st[sl]
        pltpu.async_copy(t.at[six.at[0]], tb, gsem)
        pltpu.async_copy(sbf_e.at[aix.at[0]], sb, gsem)

    def proc_grp(dix, six, aix, tb, sb, gsem):
        pltpu.make_async_copy(t.at[six.at[0]], tb, gsem).wait()
        pltpu.make_async_copy(sbf_e.at[aix.at[0]], sb, gsem).wait()

        def mulrow(r, _):
            for u in range(MUR):
                for v2 in range(INT // L):
                    sl2 = pl.ds(v2 * L, L)
                    tb[r * MUR + u, sl2] = (
                        tb[r * MUR + u, sl2] * sb[r * MUR + u, sl2])
            return 0
        lax.fori_loop(0, G // MUR, mulrow, 0)
        pltpu.sync_copy(tb, table.at[dix.at[0]], add=True)

    slot0 = (dst_ix0, src_ix0, ang_ix0, tbuf0, sbuf0, gsem0)
    slot1 = (dst_ix1, src_ix1, ang_ix1, tbuf1, sbuf1, gsem1)

    def passbody(p, _):
        stripe = CAP // NS
        lo = c * HALF + p * CAP
        hi = lo + CAP
        with jax.named_scope("sc_zero"):
            pltpu.sync_copy(zc.at[pl.ds(0, stripe)],
                            table.at[pl.ds(s * stripe, stripe)])
            plsc.subcore_barrier()

        def do_block(kb, a0blk, a1blk):
            def filt(i, cnt):
                av = a0blk[pl.ds(i * L, L)]
                a1v = a1blk[pl.ds(i * L, L)]
                m = (av >= lo) & (av < hi)
                mi = m.astype(jnp.int32)
                ic = plsc.cumsum(mi)
                pos = cnt + ic - mi
                plsc.store_scatter(dst_st, [pos], av - lo, mask=m)
                plsc.store_scatter(src_st, [pos], a1v, mask=m)
                kv = kb + i * L + lax.iota(jnp.int32, L)
                plsc.store_scatter(ang_st, [pos], kv, mask=m)
                return cnt + jnp.sum(mi)

            with jax.named_scope("sc_filt"):
                cnt = lax.fori_loop(0, NFILT, filt, 0)
            npair = (jnp.maximum(cnt, 1) + 2 * G - 1) // (2 * G)
            ngrp2 = 2 * npair
            npad = (ngrp2 * G - cnt + L - 1) // L
            iw = lax.iota(jnp.int32, L)

            def pad(w, _):
                off = cnt + w * L
                spread = (off + iw) & 255
                dst_st[pl.ds(off, L)] = DUMP + spread
                src_st[pl.ds(off, L)] = spread
                ang_st[pl.ds(off, L)] = spread
                return 0
            lax.fori_loop(0, npad, pad, 0)

            with jax.named_scope("sc_grp"):
                fire_grp(0, *slot0)
                fire_grp(1, *slot1)

                def pairbody(q, _):
                    proc_grp(*slot0)
                    fire_grp(2 * q + 2, *slot0)
                    proc_grp(*slot1)
                    fire_grp(2 * q + 3, *slot1)
                    return 0
                lax.fori_loop(0, npair - 1, pairbody, 0)
                proc_grp(*slot0)
                proc_grp(*slot1)

        npairs = NBLK // 2
        fire_idx(0, a0b0, a1b0, isem0)
        fire_idx(1, a0b1, a1b1, isem1)

        def blockpair(q, _):
            b0 = 2 * q
            wait_idx(a0b0, a1b0, isem0)
            do_block(kbase + b0 * BA, a0b0, a1b0)
            fire_idx(b0 + 2, a0b0, a1b0, isem0)
            wait_idx(a0b1, a1b1, isem1)
            do_block(kbase + (b0 + 1) * BA, a0b1, a1b1)
            fire_idx(b0 + 3, a0b1, a1b1, isem1)
            return 0
        lax.fori_loop(0, npairs - 1, blockpair, 0)
        wait_idx(a0b0, a1b0, isem0)
        do_block(kbase + (NBLK - 2) * BA, a0b0, a1b0)
        wait_idx(a0b1, a1b1, isem1)
        do_block(kbase + (NBLK - 1) * BA, a0b1, a1b1)

        with jax.named_scope("sc_out"):
            plsc.subcore_barrier()
            pltpu.sync_copy(table.at[pl.ds(s * stripe, stripe)],
                            seg.at[pl.ds(lo + s * stripe, stripe)])
            plsc.subcore_barrier()
        return 0

    lax.fori_loop(0, NP, passbody, 0)


_sc_segment = pl.kernel(
    _sc_body,
    out_type=jax.ShapeDtypeStruct((M, INT), jnp.float32),
    mesh=plsc.VectorSubcoreMesh(core_axis_name="c", subcore_axis_name="s",
                                num_cores=NC, num_subcores=NS),
    scratch_types=(
        [pltpu.VMEM((BA,), jnp.int32)] * 4
        + [pltpu.VMEM((BA + 2 * G + L,), jnp.int32)] * 3
        + [pltpu.VMEM((1, G), jnp.int32)] * 6
        + [pltpu.VMEM((G, INT), jnp.float32)] * 4
        + [pltpu.VMEM_SHARED((CAP + 256, INT), jnp.float32)]
        + [pltpu.SemaphoreType.DMA] * 4
    ),
    compiler_params=pltpu.CompilerParams(needs_layout_passes=False,
                                         use_tc_tiling_on_sc=False),
)


def kernel(x, rbf, sbf, angle_index, params):
    p = params
    w_rbf = jnp.dot(p['W_rbf1'], p['W_rbf2'], preferred_element_type=jnp.float32)
    w_sbf = jnp.dot(p['W_sbf1'], p['W_sbf2'], preferred_element_type=jnp.float32)
    x_ji, t = _stage1(x, rbf, p['W_ji'], p['b_ji'].reshape(1, EMB),
                      p['W_kj'], p['b_kj'].reshape(1, EMB), w_rbf, p['W_down'])
    sbf_e = _stage2(sbf, w_sbf)
    zc = jnp.zeros((ZSTRIPE, INT), jnp.float32)
    seg = _sc_segment(angle_index[0], angle_index[1], t, sbf_e, zc)
    return _stage3(x, x_ji, seg, p)


# speedup vs baseline: 1.0265x; 1.0265x over previous
import functools

import jax
import jax.numpy as jnp
from jax import lax
from jax.experimental import pallas as pl
from jax.experimental.pallas import tpu as pltpu
from jax.experimental.pallas import tpu_sc as plsc

EMB = 128
INT = 64
M = 160000
K = 320000

BM = 2000
BK = 4000


def _silu(v):
    return v * jax.nn.sigmoid(v)


def _dotb(a, b):
    return jnp.dot(a.astype(jnp.bfloat16), b.astype(jnp.bfloat16),
                   preferred_element_type=jnp.float32)


def _stage1_body(x_ref, rbf_ref, wji_ref, bji_ref, wkj_ref, bkj_ref,
                 wrbf_ref, wdown_ref, xji_out, t_out):
    xb = x_ref[...]
    x_ji = _silu(jnp.dot(xb, wji_ref[...], preferred_element_type=jnp.float32)
                 + bji_ref[...])
    x_kj = _silu(jnp.dot(xb, wkj_ref[...], preferred_element_type=jnp.float32)
                 + bkj_ref[...])
    rbf_e = jnp.dot(rbf_ref[...], wrbf_ref[...],
                    preferred_element_type=jnp.float32)
    x_kj = x_kj * rbf_e
    t = _silu(jnp.dot(x_kj, wdown_ref[...], preferred_element_type=jnp.float32))
    xji_out[...] = x_ji
    t_out[...] = t


def _stage1(x, rbf, w_ji, b_ji, w_kj, b_kj, w_rbf, w_down):
    grid = (M // BM,)
    blk = lambda d: pl.BlockSpec((BM, d), lambda i: (i, 0))
    full = lambda a: pl.BlockSpec(a.shape, lambda i: (0,) * a.ndim)
    return pl.pallas_call(
        _stage1_body,
        grid=grid,
        in_specs=[blk(EMB), blk(rbf.shape[1]), full(w_ji), full(b_ji),
                  full(w_kj), full(b_kj), full(w_rbf), full(w_down)],
        out_specs=[blk(EMB), blk(INT)],
        out_shape=[jax.ShapeDtypeStruct((M, EMB), jnp.float32),
                   jax.ShapeDtypeStruct((M, INT), jnp.float32)],
    )(x, rbf, w_ji, b_ji, w_kj, b_kj, w_rbf, w_down)


def _stage2_body(sbf_ref, wsbf_ref, out_ref):
    out_ref[...] = jnp.dot(sbf_ref[...], wsbf_ref[...],
                           preferred_element_type=jnp.float32)


def _stage2(sbf, w_sbf):
    grid = (K // BK,)
    return pl.pallas_call(
        _stage2_body,
        grid=grid,
        in_specs=[pl.BlockSpec((BK, sbf.shape[1]), lambda i: (i, 0)),
                  pl.BlockSpec(w_sbf.shape, lambda i: (0, 0))],
        out_specs=pl.BlockSpec((BK, INT), lambda i: (i, 0)),
        out_shape=jax.ShapeDtypeStruct((K, INT), jnp.float32),
    )(sbf, w_sbf)


def _stage3_body(x_ref, xji_ref, seg_ref, wup_ref,
                 w1_ref, b1_ref, w2_ref, b2_ref, wf_ref, bf_ref,
                 wa01_ref, ba01_ref, wa02_ref, ba02_ref,
                 wa11_ref, ba11_ref, wa12_ref, ba12_ref, out_ref):
    dot = lambda a, b: jnp.dot(a, b, preferred_element_type=jnp.float32)
    u = _silu(dot(seg_ref[...], wup_ref[...]))
    x2 = xji_ref[...] + u
    h = _silu(dot(x2, w1_ref[...]) + b1_ref[...])
    h = _silu(dot(h, w2_ref[...]) + b2_ref[...])
    x2 = x2 + h
    x2 = _silu(dot(x2, wf_ref[...]) + bf_ref[...])
    out = x_ref[...] + x2
    h = _silu(dot(out, wa01_ref[...]) + ba01_ref[...])
    h = _silu(dot(h, wa02_ref[...]) + ba02_ref[...])
    out = out + h
    h = _silu(dot(out, wa11_ref[...]) + ba11_ref[...])
    h = _silu(dot(h, wa12_ref[...]) + ba12_ref[...])
    out_ref[...] = out + h


def _stage3(x, x_ji, seg, p):
    grid = (M // BM,)
    blk = lambda d: pl.BlockSpec((BM, d), lambda i: (i, 0))
    full = lambda a: pl.BlockSpec(a.shape, lambda i: (0,) * a.ndim)
    b = lambda name: p[name].reshape(1, EMB)
    args = (x, x_ji, seg, p['W_up'],
            p['W_bs0_1'], b('b_bs0_1'), p['W_bs0_2'], b('b_bs0_2'),
            p['W_fbs'], b('b_fbs'),
            p['W_as0_1'], b('b_as0_1'), p['W_as0_2'], b('b_as0_2'),
            p['W_as1_1'], b('b_as1_1'), p['W_as1_2'], b('b_as1_2'))
    return pl.pallas_call(
        _stage3_body,
        grid=grid,
        in_specs=[blk(EMB), blk(EMB), blk(INT)] + [full(a) for a in args[3:]],
        out_specs=blk(EMB),
        out_shape=jax.ShapeDtypeStruct((M, EMB), jnp.float32),
    )(*args)


NC = 2
NS = 16
L = 16
HALF = M // NC
CAP = 20000
NP = HALF // CAP
KS = K // NS
BA = 2000
NBLK = KS // BA
NFILT = BA // L
G = 128
GV = G // L
MUR = 4
ZSTRIPE = CAP // NS
DUMP = CAP


def _sc_body(a0, a1, t, sbf_e, zc, seg,
             a0b0, a1b0, a0b1, a1b1, dst_st, src_st, ang_st,
             dst_ix0, src_ix0, ang_ix0, dst_ix1, src_ix1, ang_ix1,
             tbuf0, sbuf0, tbuf1, sbuf1, table,
             isem0, isem1, gsem0, gsem1):
    c = lax.axis_index("c")
    s = lax.axis_index("s")
    kbase = s * KS

    def fire_idx(b, a0blk, a1blk, isem):
        kb = kbase + b * BA
        pltpu.async_copy(a0.at[pl.ds(kb, BA)], a0blk, isem)
        pltpu.async_copy(a1.at[pl.ds(kb, BA)], a1blk, isem)

    def wait_idx(a0blk, a1blk, isem):
        pltpu.make_async_copy(a0.at[pl.ds(0, BA)], a0blk, isem).wait()
        pltpu.make_async_copy(a1.at[pl.ds(0, BA)], a1blk, isem).wait()

    def fire_grp(j, dix, six, aix, tb, sb, gsem):
        for v in range(GV):
            sl = pl.ds(j * G + v * L, L)
            dix[0, pl.ds(v * L, L)] = dst_st[sl]
            six[0, pl.ds(v * L, L)] = src_st[sl]
            aix[0, pl.ds(v * L, L)] = ang_st[sl]
        pltpu.async_copy(t.at[six.at[0]], tb, gsem)
        pltpu.async_copy(sbf_e.at[aix.at[0]], sb, gsem)

    def proc_grp(dix, six, aix, tb, sb, gsem):
        pltpu.make_async_copy(t.at[six.at[0]], tb, gsem).wait()
        pltpu.make_async_copy(sbf_e.at[aix.at[0]], sb, gsem).wait()

        def mulrow(r, _):
            for u in range(MUR):
                for v2 in range(INT // L):
                    sl2 = pl.ds(v2 * L, L)
                    tb[r * MUR + u, sl2] = (
                        tb[r * MUR + u, sl2] * sb[r * MUR + u, sl2])
            return 0
        lax.fori_loop(0, G // MUR, mulrow, 0)
        pltpu.sync_copy(tb, table.at[dix.at[0]], add=True)

    slot0 = (dst_ix0, src_ix0, ang_ix0, tbuf0, sbuf0, gsem0)
    slot1 = (dst_ix1, src_ix1, ang_ix1, tbuf1, sbuf1, gsem1)

    def passbody(p, _):
        stripe = CAP // NS
        lo = c * HALF + p * CAP
        hi = lo + CAP
        with jax.named_scope("sc_zero"):
            pltpu.sync_copy(zc.at[pl.ds(0, stripe)],
                            table.at[pl.ds(s * stripe, stripe)])
            plsc.subcore_barrier()

        def do_block(kb, a0blk, a1blk):
            def filt(i, cnt):
                av = a0blk[pl.ds(i * L, L)]
                a1v = a1blk[pl.ds(i * L, L)]
                m = (av >= lo) & (av < hi)
                mi = m.astype(jnp.int32)
                ic = plsc.cumsum(mi)
                pos = cnt + ic - mi
                plsc.store_scatter(dst_st, [pos], av - lo, mask=m)
                plsc.store_scatter(src_st, [pos], a1v, mask=m)
                kv = kb + i * L + lax.iota(jnp.int32, L)
                plsc.store_scatter(ang_st, [pos], kv, mask=m)
                return cnt + ic[L - 1]

            with jax.named_scope("sc_filt"):
                cnt = lax.fori_loop(0, NFILT, filt, 0)
            npair = (jnp.maximum(cnt, 1) + 2 * G - 1) // (2 * G)
            ngrp2 = 2 * npair
            npad = (ngrp2 * G - cnt + L - 1) // L
            iw = lax.iota(jnp.int32, L)

            def pad(w, _):
                off = cnt + w * L
                spread = (off + iw) & 255
                dst_st[pl.ds(off, L)] = DUMP + spread
                src_st[pl.ds(off, L)] = spread
                ang_st[pl.ds(off, L)] = spread
                return 0
            lax.fori_loop(0, npad, pad, 0)

            with jax.named_scope("sc_grp"):
                fire_grp(0, *slot0)
                fire_grp(1, *slot1)

                def pairbody(q, _):
                    proc_grp(*slot0)
                    fire_grp(2 * q + 2, *slot0)
                    proc_grp(*slot1)
                    fire_grp(2 * q + 3, *slot1)
                    return 0
                lax.fori_loop(0, npair - 1, pairbody, 0)
                proc_grp(*slot0)
                proc_grp(*slot1)

        npairs = NBLK // 2
        fire_idx(0, a0b0, a1b0, isem0)
        fire_idx(1, a0b1, a1b1, isem1)

        def blockpair(q, _):
            b0 = 2 * q
            wait_idx(a0b0, a1b0, isem0)
            do_block(kbase + b0 * BA, a0b0, a1b0)
            fire_idx(b0 + 2, a0b0, a1b0, isem0)
            wait_idx(a0b1, a1b1, isem1)
            do_block(kbase + (b0 + 1) * BA, a0b1, a1b1)
            fire_idx(b0 + 3, a0b1, a1b1, isem1)
            return 0
        lax.fori_loop(0, npairs - 1, blockpair, 0)
        wait_idx(a0b0, a1b0, isem0)
        do_block(kbase + (NBLK - 2) * BA, a0b0, a1b0)
        wait_idx(a0b1, a1b1, isem1)
        do_block(kbase + (NBLK - 1) * BA, a0b1, a1b1)

        with jax.named_scope("sc_out"):
            plsc.subcore_barrier()
            pltpu.sync_copy(table.at[pl.ds(s * stripe, stripe)],
                            seg.at[pl.ds(lo + s * stripe, stripe)])
            plsc.subcore_barrier()
        return 0

    lax.fori_loop(0, NP, passbody, 0)


_sc_segment = pl.kernel(
    _sc_body,
    out_type=jax.ShapeDtypeStruct((M, INT), jnp.float32),
    mesh=plsc.VectorSubcoreMesh(core_axis_name="c", subcore_axis_name="s",
                                num_cores=NC, num_subcores=NS),
    scratch_types=(
        [pltpu.VMEM((BA,), jnp.int32)] * 4
        + [pltpu.VMEM((BA + 2 * G + L,), jnp.int32)] * 3
        + [pltpu.VMEM((1, G), jnp.int32)] * 6
        + [pltpu.VMEM((G, INT), jnp.float32)] * 4
        + [pltpu.VMEM_SHARED((CAP + 256, INT), jnp.float32)]
        + [pltpu.SemaphoreType.DMA] * 4
    ),
    compiler_params=pltpu.CompilerParams(needs_layout_passes=False,
                                         use_tc_tiling_on_sc=False),
)


def kernel(x, rbf, sbf, angle_index, params):
    p = params
    w_rbf = jnp.dot(p['W_rbf1'], p['W_rbf2'], preferred_element_type=jnp.float32)
    w_sbf = jnp.dot(p['W_sbf1'], p['W_sbf2'], preferred_element_type=jnp.float32)
    x_ji, t = _stage1(x, rbf, p['W_ji'], p['b_ji'].reshape(1, EMB),
                      p['W_kj'], p['b_kj'].reshape(1, EMB), w_rbf, p['W_down'])
    sbf_e = _stage2(sbf, w_sbf)
    zc = jnp.zeros((ZSTRIPE, INT), jnp.float32)
    seg = _sc_segment(angle_index[0], angle_index[1], t, sbf_e, zc)
    return _stage3(x, x_ji, seg, p)
